# Initial kernel scaffold; baseline (speedup 1.0000x reference)
import jax
import jax.numpy as jnp
from jax.experimental import pallas as pl


def kernel(X, edge_index, W_w, b_w, W_a, b_a, W_r, b_r):
    raise NotImplementedError("write your pallas kernel here")


# speedup vs baseline: 5.0130x; 5.0130x over previous
import functools

import jax
import jax.numpy as jnp
from jax import lax
from jax.experimental import pallas as pl
from jax.experimental.pallas import tpu as pltpu
from jax.experimental.pallas import tpu_sc as plsc

N = 10000
NP = 10240
E = 160000
B = 2
D = 128
DP = 144
ALPHA = 0.2

NS = 16
E_T = E // NS
CH = 80
NCH = E_T // CH
N_T = NP // NS
FCH = CH
NFC = N_T // FCH


def _lrelu(x):
    return jnp.where(x >= 0, x, ALPHA * x)


def _dense_body(x_ref, ww_ref, bw_ref, wa_ref, ba_ref, wr_ref, br_ref,
                whp_ref, s1_ref, res_ref):
    x = x_ref[0]
    wh = lax.dot_general(x, ww_ref[...], (((1,), (1,)), ((), ())),
                         preferred_element_type=jnp.float32) + bw_ref[0][None, :]
    wh = _lrelu(wh)
    res = lax.dot_general(x, wr_ref[...], (((1,), (1,)), ((), ())),
                          preferred_element_type=jnp.float32) + br_ref[0][None, :]
    wa = wa_ref[...]
    a1 = wa[:, :D]
    a2 = wa[:, D:]
    s1 = jnp.sum(wh * a1, axis=1) + ba_ref[0, 0]
    s2 = jnp.sum(wh * a2, axis=1)
    res_ref[0, :N] = res
    whp_ref[0, :N, :D] = wh
    lane16 = lax.broadcasted_iota(jnp.int32, (x.shape[0], DP - D), 1)
    whp_ref[0, :N, D:] = jnp.where(lane16 == 0, s2[:, None], 0.0)
    s1_ref[0, 0] = s1


def _dense(X, W_w, b_w, W_a, b_a, W_r, b_r):
    full = lambda shape: pl.BlockSpec(shape, lambda b: (0,) * len(shape))
    return pl.pallas_call(
        _dense_body,
        grid=(B,),
        in_specs=[
            pl.BlockSpec((1, N, D), lambda b: (b, 0, 0)),
            full((D, D)),
            full((1, D)),
            full((1, 2 * D)),
            full((1, 1)),
            full((D, D)),
            full((1, D)),
        ],
        out_specs=[
            pl.BlockSpec((1, NP, DP), lambda b: (b, 0, 0)),
            pl.BlockSpec((1, 1, N), lambda b: (b, 0, 0)),
            pl.BlockSpec((1, NP, D), lambda b: (b, 0, 0)),
        ],
        out_shape=[
            jax.ShapeDtypeStruct((B, NP, DP), jnp.float32),
            jax.ShapeDtypeStruct((B, 1, N), jnp.float32),
            jax.ShapeDtypeStruct((B, NP, D), jnp.float32),
        ],
    )(X, W_w, b_w, W_a, b_a, W_r, b_r)


CAP = 12000
NCHP = CAP // CH
SENT = 1023
RSP = 672


def _sc_body(whp, s1h, srelh, dsth, resh, outh,
             s1v, srelv, acc, rs, buf, idxb, rbuf, sem):
    b = lax.axis_index("c")
    tid = lax.axis_index("s")
    lo = tid * N_T

    pltpu.sync_copy(s1h.at[pl.ds(b * N, N)], s1v)
    pltpu.sync_copy(srelh.at[pl.ds(tid * CAP, CAP)], srelv)

    lanes = lax.broadcasted_iota(jnp.int32, (16,), 0)
    zero16 = jnp.zeros((16,), jnp.float32)
    ones16 = jnp.ones((16,), jnp.float32)

    def _zacc(i, c):
        for cc in range(D // 16):
            acc[i, pl.ds(cc * 16, 16)] = zero16
        return c
    lax.fori_loop(0, N_T, _zacc, 0)
    def _zrs(i, c):
        rs[pl.ds(i * 16, 16)] = zero16
        return c
    lax.fori_loop(0, RSP // 16, _zrs, 0)

    col_e = jnp.full((16,), D, jnp.int32)
    onehot = jnp.where(lanes == 0, 1.0, 0.0)
    dbase = (b * NS + tid) * CAP
    def _chunk(p, c):
        pltpu.sync_copy(dsth.at[pl.ds(dbase + p * CH, CH)], idxb)
        pltpu.async_copy(whp.at[idxb], buf, sem).wait()
        def _grp(g, c2):
            rows = g * 16 + lanes
            srel = srelv[pl.ds(p * CH + g * 16, 16)]
            valid = srel < N_T
            srel0 = jnp.where(valid, srel, 0)
            s1g = plsc.load_gather(s1v, [jnp.where(valid, srel0 + lo, 0)])
            s2g = plsc.load_gather(buf, [rows, col_e])
            e = jnp.exp(-_lrelu(s1g + s2g))
            w16 = jnp.where(valid, e, 0.0)
            for i in range(16):
                w = w16[i]
                r = srel0[i]
                rbl = g * 16 + i
                for cc in range(D // 16):
                    sl = pl.ds(cc * 16, 16)
                    acc[r, sl] = acc[r, sl] + buf[rbl, sl] * w
                rsl = pl.ds(r, 16)
                rs[rsl] = rs[rsl] + w * onehot
            return c2
        lax.fori_loop(0, CH // 16, _grp, 0)
        return c
    lax.fori_loop(0, NCHP, _chunk, 0)

    eps16 = jnp.full((16,), 9e-15, jnp.float32)
    def _fin(k, c):
        nb0 = b * NP + lo + k * 16
        pltpu.sync_copy(resh.at[pl.ds(nb0, 16)], rbuf)
        for i in range(16):
            row = k * 16 + i
            inv = (ones16 / (rs[pl.ds(row, 16)] + eps16))[0]
            for cc in range(D // 16):
                sl = pl.ds(cc * 16, 16)
                hv = acc[row, sl] * inv + rbuf[i, sl]
                rbuf[i, sl] = jnp.where(hv >= 0, hv, ALPHA * hv)
        pltpu.sync_copy(rbuf, outh.at[pl.ds(nb0, 16)])
        return c
    lax.fori_loop(0, N_T // 16, _fin, 0)


@functools.partial(
    pl.kernel,
    out_type=jax.ShapeDtypeStruct((B * NP, D), jnp.float32),
    mesh=plsc.VectorSubcoreMesh(core_axis_name="c", subcore_axis_name="s",
                                num_cores=2, num_subcores=16),
    compiler_params=pltpu.CompilerParams(needs_layout_passes=False,
                                         use_tc_tiling_on_sc=False),
    scratch_types=[
        pltpu.VMEM((N,), jnp.float32),
        pltpu.VMEM((CAP,), jnp.int32),
        pltpu.VMEM((N_T, D), jnp.float32),
        pltpu.VMEM((RSP,), jnp.float32),
        pltpu.VMEM((CH, DP), jnp.float32),
        pltpu.VMEM((CH,), jnp.int32),
        pltpu.VMEM((16, D), jnp.float32),
        pltpu.SemaphoreType.DMA,
    ],
)
def _sc_aggregate(whp, s1h, srelh, dsth, resh, outh, *rest):
    _sc_body(whp, s1h, srelh, dsth, resh, outh, *rest)


def kernel(X, edge_index, W_w, b_w, W_a, b_a, W_r, b_r):
    whp, s1, res = _dense(X, W_w, b_w.reshape(1, D), W_a,
                          b_a.reshape(1, 1), W_r, b_r.reshape(1, D))
    src, dst = edge_index[0], edge_index[1]
    owner = src // N_T
    order = jnp.argsort(owner)
    so, ss, sd = owner[order], src[order], dst[order]
    starts = jnp.concatenate([jnp.zeros((1,), jnp.int32),
                              jnp.cumsum(jnp.bincount(owner, length=NS))[:-1]
                              .astype(jnp.int32)])
    offs = jnp.arange(E, dtype=jnp.int32) - starts[so]
    srel_p = jnp.full((NS, CAP), SENT, jnp.int32).at[so, offs]         .set(ss - so * N_T, mode="drop")
    dst_p = jnp.zeros((NS, CAP), jnp.int32).at[so, offs].set(sd, mode="drop")
    dst_p2 = jnp.stack([dst_p, dst_p + NP]).reshape(-1)
    out = _sc_aggregate(whp.reshape(B * NP, DP), s1.reshape(B * N),
                        srel_p.reshape(-1), dst_p2,
                        res.reshape(B * NP, D))
    return out.reshape(B, NP, D)[:, :N]


# speedup vs baseline: 5.4846x; 1.0941x over previous
import functools

import jax
import jax.numpy as jnp
from jax import lax
from jax.experimental import pallas as pl
from jax.experimental.pallas import tpu as pltpu
from jax.experimental.pallas import tpu_sc as plsc

N = 10000
NP = 10240
E = 160000
B = 2
D = 128
DP = 144
ALPHA = 0.2

NS = 16
E_T = E // NS
CH = 80
NCH = E_T // CH
N_T = NP // NS
FCH = CH
NFC = N_T // FCH


def _lrelu(x):
    return jnp.where(x >= 0, x, ALPHA * x)


def _dense_body(x_ref, ww_ref, bw_ref, wa_ref, ba_ref, wr_ref, br_ref,
                whp_ref, s1_ref, res_ref):
    x = x_ref[0]
    wh = lax.dot_general(x, ww_ref[...], (((1,), (1,)), ((), ())),
                         preferred_element_type=jnp.float32) + bw_ref[0][None, :]
    wh = _lrelu(wh)
    res = lax.dot_general(x, wr_ref[...], (((1,), (1,)), ((), ())),
                          preferred_element_type=jnp.float32) + br_ref[0][None, :]
    wa = wa_ref[...]
    a1 = wa[:, :D]
    a2 = wa[:, D:]
    s1 = jnp.sum(wh * a1, axis=1) + ba_ref[0, 0]
    s2 = jnp.sum(wh * a2, axis=1)
    res_ref[0, :N] = res
    whp_ref[0, :N, :D] = wh
    lane16 = lax.broadcasted_iota(jnp.int32, (x.shape[0], DP - D), 1)
    whp_ref[0, :N, D:] = jnp.where(lane16 == 0, s2[:, None], 0.0)
    s1_ref[0, 0] = s1


def _dense(X, W_w, b_w, W_a, b_a, W_r, b_r):
    full = lambda shape: pl.BlockSpec(shape, lambda b: (0,) * len(shape))
    return pl.pallas_call(
        _dense_body,
        grid=(B,),
        in_specs=[
            pl.BlockSpec((1, N, D), lambda b: (b, 0, 0)),
            full((D, D)),
            full((1, D)),
            full((1, 2 * D)),
            full((1, 1)),
            full((D, D)),
            full((1, D)),
        ],
        out_specs=[
            pl.BlockSpec((1, NP, DP), lambda b: (b, 0, 0)),
            pl.BlockSpec((1, 1, N), lambda b: (b, 0, 0)),
            pl.BlockSpec((1, NP, D), lambda b: (b, 0, 0)),
        ],
        out_shape=[
            jax.ShapeDtypeStruct((B, NP, DP), jnp.float32),
            jax.ShapeDtypeStruct((B, 1, N), jnp.float32),
            jax.ShapeDtypeStruct((B, NP, D), jnp.float32),
        ],
    )(X, W_w, b_w, W_a, b_a, W_r, b_r)


CAP = 12000
NCHP = CAP // CH
SENT = 1023
RSP = 672


def _sc_body(whp, s1h, srelh, dsth, resh, outh,
             s1v, srelv, acc, rs, buf, buf2, idxb, idxb2, rbuf,
             isem, isem2, gsem, gsem2):
    b = lax.axis_index("c")
    tid = lax.axis_index("s")
    lo = tid * N_T

    pltpu.sync_copy(s1h.at[pl.ds(b * N, N)], s1v)
    pltpu.sync_copy(srelh.at[pl.ds(tid * CAP, CAP)], srelv)

    lanes = lax.broadcasted_iota(jnp.int32, (16,), 0)
    zero16 = jnp.zeros((16,), jnp.float32)
    ones16 = jnp.ones((16,), jnp.float32)

    def _zacc(i, c):
        for cc in range(D // 16):
            acc[i, pl.ds(cc * 16, 16)] = zero16
        return c
    lax.fori_loop(0, N_T, _zacc, 0)
    def _zrs(i, c):
        rs[pl.ds(i * 16, 16)] = zero16
        return c
    lax.fori_loop(0, RSP // 16, _zrs, 0)

    col_e = jnp.full((16,), D, jnp.int32)
    onehot = jnp.where(lanes == 0, 1.0, 0.0)
    dbase = (b * NS + tid) * CAP

    idxbs = (idxb, idxb2)
    bufs = (buf, buf2)
    isems = (isem, isem2)
    gsems = (gsem, gsem2)

    def _start_idx(p, q):
        pltpu.async_copy(dsth.at[pl.ds(dbase + p * CH, CH)], idxbs[q], isems[q])

    def _start_gather(q):
        pltpu.async_copy(whp.at[idxbs[q]], bufs[q], gsems[q])

    def _compute(p, q):
        bq = bufs[q]
        def _grp(g, c2):
            rows = g * 16 + lanes
            srel = srelv[pl.ds(p * CH + g * 16, 16)]
            valid = srel < N_T
            srel0 = jnp.where(valid, srel, 0)
            s1g = plsc.load_gather(s1v, [jnp.where(valid, srel0 + lo, 0)])
            s2g = plsc.load_gather(bq, [rows, col_e])
            e = jnp.exp(-_lrelu(s1g + s2g))
            w16 = jnp.where(valid, e, 0.0)
            for i in range(16):
                w = w16[i]
                r = srel0[i]
                rbl = g * 16 + i
                for cc in range(D // 16):
                    sl = pl.ds(cc * 16, 16)
                    acc[r, sl] = acc[r, sl] + bq[rbl, sl] * w
                rsl = pl.ds(r, 16)
                rs[rsl] = rs[rsl] + w * onehot
            return c2
        lax.fori_loop(0, CH // 16, _grp, 0)

    def _wait_idx(q):
        pltpu.make_async_copy(dsth.at[pl.ds(dbase, CH)], idxbs[q], isems[q]).wait()

    def _wait_gather(q):
        pltpu.make_async_copy(whp.at[idxbs[q]], bufs[q], gsems[q]).wait()

    _start_idx(0, 0)
    _wait_idx(0)
    _start_gather(0)
    _start_idx(1, 1)

    def _pair(k, c):
        p = 2 * k
        _wait_gather(0)
        _wait_idx(1)
        _start_gather(1)
        _start_idx(p + 2, 0)
        _compute(p, 0)
        _wait_gather(1)
        _wait_idx(0)
        _start_gather(0)
        _start_idx(p + 3, 1)
        _compute(p + 1, 1)
        return c
    lax.fori_loop(0, NCHP // 2 - 1, _pair, 0)

    _wait_gather(0)
    _wait_idx(1)
    _start_gather(1)
    _compute(NCHP - 2, 0)
    _wait_gather(1)
    _compute(NCHP - 1, 1)

    eps16 = jnp.full((16,), 9e-15, jnp.float32)
    def _fin(k, c):
        nb0 = b * NP + lo + k * 16
        pltpu.sync_copy(resh.at[pl.ds(nb0, 16)], rbuf)
        for i in range(16):
            row = k * 16 + i
            inv = (ones16 / (rs[pl.ds(row, 16)] + eps16))[0]
            for cc in range(D // 16):
                sl = pl.ds(cc * 16, 16)
                hv = acc[row, sl] * inv + rbuf[i, sl]
                rbuf[i, sl] = jnp.where(hv >= 0, hv, ALPHA * hv)
        pltpu.sync_copy(rbuf, outh.at[pl.ds(nb0, 16)])
        return c
    lax.fori_loop(0, N_T // 16, _fin, 0)


@functools.partial(
    pl.kernel,
    out_type=jax.ShapeDtypeStruct((B * NP, D), jnp.float32),
    mesh=plsc.VectorSubcoreMesh(core_axis_name="c", subcore_axis_name="s",
                                num_cores=2, num_subcores=16),
    compiler_params=pltpu.CompilerParams(needs_layout_passes=False,
                                         use_tc_tiling_on_sc=False),
    scratch_types=[
        pltpu.VMEM((N,), jnp.float32),
        pltpu.VMEM((CAP,), jnp.int32),
        pltpu.VMEM((N_T, D), jnp.float32),
        pltpu.VMEM((RSP,), jnp.float32),
        pltpu.VMEM((CH, DP), jnp.float32),
        pltpu.VMEM((CH, DP), jnp.float32),
        pltpu.VMEM((CH,), jnp.int32),
        pltpu.VMEM((CH,), jnp.int32),
        pltpu.VMEM((16, D), jnp.float32),
        pltpu.SemaphoreType.DMA,
        pltpu.SemaphoreType.DMA,
        pltpu.SemaphoreType.DMA,
        pltpu.SemaphoreType.DMA,
    ],
)
def _sc_aggregate(whp, s1h, srelh, dsth, resh, outh, *rest):
    _sc_body(whp, s1h, srelh, dsth, resh, outh, *rest)


def kernel(X, edge_index, W_w, b_w, W_a, b_a, W_r, b_r):
    whp, s1, res = _dense(X, W_w, b_w.reshape(1, D), W_a,
                          b_a.reshape(1, 1), W_r, b_r.reshape(1, D))
    src, dst = edge_index[0], edge_index[1]
    owner = src // N_T
    order = jnp.argsort(owner)
    so, ss, sd = owner[order], src[order], dst[order]
    starts = jnp.concatenate([jnp.zeros((1,), jnp.int32),
                              jnp.cumsum(jnp.bincount(owner, length=NS))[:-1]
                              .astype(jnp.int32)])
    offs = jnp.arange(E, dtype=jnp.int32) - starts[so]
    srel_p = jnp.full((NS, CAP), SENT, jnp.int32).at[so, offs]         .set(ss - so * N_T, mode="drop")
    dst_p = jnp.zeros((NS, CAP), jnp.int32).at[so, offs].set(sd, mode="drop")
    dst_p2 = jnp.stack([dst_p, dst_p + NP]).reshape(-1)
    out = _sc_aggregate(whp.reshape(B * NP, DP), s1.reshape(B * N),
                        srel_p.reshape(-1), dst_p2,
                        res.reshape(B * NP, D))
    return out.reshape(B, NP, D)[:, :N]
